# baseline (speedup 1.0000x reference)
import functools

import jax
import jax.numpy as jnp
from jax import lax
from jax.experimental import pallas as pl
from jax.experimental.pallas import tpu as pltpu
from jax.experimental.pallas import tpu_sc as plsc

_TIME_INTERVAL = 10000.0
_N_TIME_INTERVAL = 100.0
_B = 4096
_L = 200
_EMB = 64
_NTOK = _B * _L


def _build(nw):
    rows_pw = _B // nw
    tok_pw = rows_pw * _L
    pair_pw = tok_pw // 2
    chunk = 512
    cpair = chunk // 2
    nchunk = tok_pw // chunk
    npairs = nchunk // 2
    tsw = _L + 1

    mesh = plsc.VectorSubcoreMesh(core_axis_name="c", subcore_axis_name="s")

    @functools.partial(
        pl.kernel,
        mesh=mesh,
        out_type=jax.ShapeDtypeStruct((_NTOK // 2, 2 * _EMB), jnp.float32),
        scratch_types=[
            pltpu.VMEM((rows_pw * tsw,), jnp.int32),
            pltpu.VMEM((tok_pw,), jnp.int32),
            pltpu.VMEM((cpair, 2 * _EMB), jnp.float32),
            pltpu.VMEM((cpair, 2 * _EMB), jnp.float32),
            pltpu.SemaphoreType.DMA,
            pltpu.SemaphoreType.DMA,
            pltpu.SemaphoreType.DMA,
            pltpu.SemaphoreType.DMA,
        ],
        compiler_params=pltpu.CompilerParams(use_tc_tiling_on_sc=False,
                                             needs_layout_passes=False),
    )
    def k(ts_hbm, table_hbm, out_hbm, ts_v, idx_v, rows0, rows1,
          gsem0, gsem1, osem0, osem1):
        sid = lax.axis_index("s")
        wid = lax.axis_index("c") * 16 + sid
        pr0 = wid * pair_pw

        pltpu.sync_copy(ts_hbm.at[pl.ds(wid * rows_pw * tsw, rows_pw * tsw)],
                        ts_v)

        def idx_body(r, carry):
            for i in range(13):
                c = 184 if i == 12 else i * 16
                t1 = ts_v[pl.ds(r * tsw + c + 1, 16)]
                t0 = ts_v[pl.ds(r * tsw + c, 16)]
                dt = (t1 - t0).astype(jnp.float32)
                bix = (dt / _TIME_INTERVAL * _N_TIME_INTERVAL).astype(jnp.int32)
                idx_v[pl.ds(r * _L + c, 16)] = jnp.maximum(bix, 0)
            return carry

        lax.fori_loop(0, rows_pw, idx_body, 0)

        iota2 = lax.iota(jnp.int32, 16) * 2

        def pair_idx_body(q, carry):
            base = q * 32
            e = plsc.load_gather(idx_v, [iota2 + base])
            o = plsc.load_gather(idx_v, [iota2 + (base + 1)])
            idx_v[pl.ds(q * 16, 16)] = e * 101 + o
            return carry

        lax.fori_loop(0, pair_pw // 16, pair_idx_body, 0)

        def gathers(g, rows, sem):
            for j in range(cpair // 128):
                pltpu.async_copy(
                    table_hbm.at[idx_v.at[pl.ds(g * cpair + j * 128, 128)]],
                    rows.at[pl.ds(j * 128, 128)],
                    sem,
                )

        def drain_gather(rows, sem):
            pltpu.make_async_copy(out_hbm.at[pl.ds(0, cpair)], rows, sem).wait()

        def drain_write(sem):
            pltpu.make_async_copy(rows0, out_hbm.at[pl.ds(0, cpair)], sem).wait()

        gathers(0, rows0, gsem0)

        def pair_body(gg, carry):
            g0 = gg * 2
            g1 = g0 + 1
            drain_gather(rows0, gsem0)

            @pl.when(gg > 0)
            def _():
                drain_write(osem1)
            gathers(g1, rows1, gsem1)
            pltpu.async_copy(rows0, out_hbm.at[pl.ds(pr0 + g0 * cpair, cpair)],
                             osem0)

            drain_gather(rows1, gsem1)
            drain_write(osem0)

            @pl.when(gg + 1 < npairs)
            def _():
                gathers(g0 + 2, rows0, gsem0)
            pltpu.async_copy(rows1, out_hbm.at[pl.ds(pr0 + g1 * cpair, cpair)],
                             osem1)
            return carry

        lax.fori_loop(0, npairs, pair_body, 0)
        drain_write(osem1)

    return k


def kernel(inputs, timestamp, W, b):
    info = plsc.get_sparse_core_info()
    nw = info.num_cores * info.num_subcores
    table = (W.T + b[None, :]).astype(jnp.float32)
    pair_table = jnp.concatenate(
        [jnp.repeat(table, 101, axis=0), jnp.tile(table, (101, 1))], axis=1)
    ts_flat = timestamp.astype(jnp.int32).reshape(-1)
    out = _build(nw)(ts_flat, pair_table)
    return out.reshape(_B, _L, _EMB)


# speedup vs baseline: 17.8190x; 17.8190x over previous
import functools

import jax
import jax.numpy as jnp
from jax import lax
from jax.experimental import pallas as pl
from jax.experimental.pallas import tpu as pltpu
from jax.experimental.pallas import tpu_sc as plsc

_TIME_INTERVAL = 10000.0
_N_TIME_INTERVAL = 100.0
_B = 4096
_L = 200
_EMB = 64
_NTOK = _B * _L


def _build(nw):
    tok_pw = _NTOK // nw
    pair_pw = tok_pw // 2
    chunk = 256
    cpair = chunk // 2
    nchunk = tok_pw // chunk
    nloop = nchunk // 2
    tblk = 2048
    cpb = tblk // chunk

    mesh = plsc.VectorSubcoreMesh(core_axis_name="c", subcore_axis_name="s")

    @functools.partial(
        pl.kernel,
        mesh=mesh,
        out_type=jax.ShapeDtypeStruct((_NTOK // 2, 2 * _EMB), jnp.float32),
        scratch_types=[
            pltpu.VMEM((tblk,), jnp.int32),
            pltpu.VMEM((tblk,), jnp.int32),
            pltpu.VMEM((chunk,), jnp.int32),
            pltpu.VMEM((cpair,), jnp.int32),
            pltpu.VMEM((cpair,), jnp.int32),
            pltpu.VMEM((cpair, 2 * _EMB), jnp.float32),
            pltpu.VMEM((cpair, 2 * _EMB), jnp.float32),
            pltpu.VMEM_SHARED((101 * 101, 2 * _EMB), jnp.float32),
            pltpu.SemaphoreType.DMA,
            pltpu.SemaphoreType.DMA,
            pltpu.SemaphoreType.DMA,
            pltpu.SemaphoreType.DMA,
        ],
        compiler_params=pltpu.CompilerParams(use_tc_tiling_on_sc=False,
                                             needs_layout_passes=False),
    )
    def k(ts0_hbm, ts1_hbm, table_hbm, out_hbm, t0_v, t1_v, tix_v,
          pidx0, pidx1, rows0, rows1, table_sh,
          gsem0, gsem1, osem0, osem1):
        sid = lax.axis_index("s")
        wid = lax.axis_index("c") * 16 + sid
        tok0 = wid * tok_pw
        pr0 = wid * pair_pw

        @pl.when(sid == 0)
        def _():
            pltpu.sync_copy(table_hbm, table_sh)

        iota2 = lax.iota(jnp.int32, 16) * 2

        def load_ts_block(blk):
            base = tok0 + blk * tblk
            pltpu.sync_copy(ts0_hbm.at[pl.ds(base, tblk)], t0_v)
            pltpu.sync_copy(ts1_hbm.at[pl.ds(base, tblk)], t1_v)

        def compute_pidx(g, pidx):
            off = (g % cpb) * chunk
            for i in range(chunk // 16):
                c = off + i * 16
                dt = (t1_v[pl.ds(c, 16)] - t0_v[pl.ds(c, 16)]).astype(jnp.float32)
                bix = (dt / _TIME_INTERVAL * _N_TIME_INTERVAL).astype(jnp.int32)
                tix_v[pl.ds(i * 16, 16)] = jnp.maximum(bix, 0)
            for q in range(cpair // 16):
                e = plsc.load_gather(tix_v, [iota2 + q * 32])
                o = plsc.load_gather(tix_v, [iota2 + (q * 32 + 1)])
                pidx[pl.ds(q * 16, 16)] = e * 101 + o

        def gather(rows, pidx, sem):
            pltpu.async_copy(table_sh.at[pidx], rows, sem)

        def drain_gather(rows, sem):
            pltpu.make_async_copy(out_hbm.at[pl.ds(0, cpair)], rows, sem).wait()

        def drain_write(sem):
            pltpu.make_async_copy(rows0, out_hbm.at[pl.ds(0, cpair)], sem).wait()

        plsc.subcore_barrier()

        load_ts_block(0)
        compute_pidx(0, pidx0)
        gather(rows0, pidx0, gsem0)

        def loop_body(gg, carry):
            g0 = gg * 2
            g1 = g0 + 1

            @pl.when((g1 % cpb) == 0)
            def _():
                load_ts_block(g1 // cpb)
            compute_pidx(g1, pidx1)
            drain_gather(rows0, gsem0)

            @pl.when(gg > 0)
            def _():
                drain_write(osem1)
            gather(rows1, pidx1, gsem1)
            pltpu.async_copy(rows0, out_hbm.at[pl.ds(pr0 + g0 * cpair, cpair)],
                             osem0)

            @pl.when(gg + 1 < nloop)
            def _():
                g2 = g0 + 2

                @pl.when((g2 % cpb) == 0)
                def _():
                    load_ts_block(g2 // cpb)
                compute_pidx(g2, pidx0)
            drain_gather(rows1, gsem1)
            drain_write(osem0)

            @pl.when(gg + 1 < nloop)
            def _():
                gather(rows0, pidx0, gsem0)
            pltpu.async_copy(rows1, out_hbm.at[pl.ds(pr0 + g1 * cpair, cpair)],
                             osem1)
            return carry

        lax.fori_loop(0, nloop, loop_body, 0)
        drain_write(osem1)

    return k


def kernel(inputs, timestamp, W, b):
    info = plsc.get_sparse_core_info()
    nw = info.num_cores * info.num_subcores
    table = (W.T + b[None, :]).astype(jnp.float32)
    pair_table = jnp.concatenate(
        [jnp.repeat(table, 101, axis=0), jnp.tile(table, (101, 1))], axis=1)
    ts = timestamp.astype(jnp.int32)
    ts0 = ts[:, :-1].reshape(-1)
    ts1 = ts[:, 1:].reshape(-1)
    out = _build(nw)(ts0, ts1, pair_table)
    return out.reshape(_B, _L, _EMB)
